# Initial kernel scaffold; baseline (speedup 1.0000x reference)
import jax
import jax.numpy as jnp
from jax.experimental import pallas as pl


def kernel(edges, n_samples, eps, mu, log_sigma):
    raise NotImplementedError("write your pallas kernel here")


# speedup vs baseline: 14.6558x; 14.6558x over previous
import functools

import jax
import jax.numpy as jnp
from jax import lax
from jax.experimental import pallas as pl
from jax.experimental.pallas import tpu as pltpu
from jax.experimental.pallas import tpu_sc as plsc

NC = 2
NS = 16
NW = NC * NS
L = 16

LN2 = 0.6931471805599453
SQRT2 = 1.4142135381698608


def _c_tc_body(mu_ref, ls_ref, eps_ref, c_ref):
    c_ref[...] = mu_ref[...] + jnp.exp(ls_ref[...]) * eps_ref[...]


def _log_f32(y):
    yi = plsc.bitcast(y, jnp.int32)
    e = lax.shift_right_logical(yi, 23) - 127
    m = plsc.bitcast((yi & 0x007FFFFF) | 0x3F800000, jnp.float32)
    big = m >= SQRT2
    m = jnp.where(big, m * 0.5, m)
    ef = (e + jnp.where(big, 1, 0)).astype(jnp.float32)
    s = (m - 1.0) / (m + 1.0)
    z = s * s
    p = s * (2.0 + z * (0.66666667 + z * (0.4 + z * 0.28571429)))
    return ef * LN2 + p


def _make_sc_loglik(n_edges, n_pad):
    assert n_edges % NW == 0
    edges_per_tile = n_edges // NW
    chunk = 4000
    while edges_per_tile % chunk:
        chunk -= 16
    n_chunks = edges_per_tile // chunk
    groups = chunk // L

    mesh = plsc.VectorSubcoreMesh(
        core_axis_name="c", subcore_axis_name="s",
        num_cores=NC, num_subcores=NS)

    @functools.partial(
        pl.kernel,
        out_type=jax.ShapeDtypeStruct((NW, L), jnp.float32),
        mesh=mesh,
        compiler_params=pltpu.CompilerParams(needs_layout_passes=False),
        scratch_types=[
            pltpu.VMEM((n_pad,), jnp.float32),
            pltpu.VMEM((2 * chunk,), jnp.int32),
            pltpu.VMEM((L,), jnp.float32),
        ],
    )
    def sc_loglik(edges_hbm, c_hbm, out_hbm, c_v, ebuf, acc_v):
        cid = lax.axis_index("c")
        sid = lax.axis_index("s")
        wid = sid * NC + cid
        pltpu.sync_copy(c_hbm, c_v)
        base_elt = wid * (2 * edges_per_tile)
        iota2 = lax.iota(jnp.int32, L) * 2

        def grp_body(g, acc):
            iv = iota2 + g * (2 * L)
            ii = plsc.load_gather(ebuf, [iv])
            jj = plsc.load_gather(ebuf, [iv + 1])
            ci = plsc.load_gather(c_v, [ii])
            cj = plsc.load_gather(c_v, [jj])
            x = ci * cj
            t = jnp.exp(-x)
            y = 1.0 / (1.0 + t) + 1e-8
            return acc + _log_f32(y)

        def chunk_body(k, acc):
            elt0 = pl.multiple_of(base_elt + k * (2 * chunk), 8)
            pltpu.sync_copy(edges_hbm.at[pl.ds(elt0, 2 * chunk)], ebuf)
            return lax.fori_loop(0, groups, grp_body, acc)

        acc = lax.fori_loop(0, n_chunks, chunk_body,
                            jnp.zeros((L,), jnp.float32))
        acc_v[...] = acc
        pltpu.sync_copy(acc_v, out_hbm.at[wid])

    return sc_loglik


def kernel(edges, n_samples, eps, mu, log_sigma):
    n = mu.shape[0]
    n_edges = edges.shape[0]
    n_pad = ((n + 1023) // 1024) * 1024
    rows = n_pad // 128
    mu2 = jnp.pad(mu, (0, n_pad - n)).reshape(rows, 128)
    ls2 = jnp.pad(log_sigma, (0, n_pad - n)).reshape(rows, 128)
    eps2 = jnp.pad(eps[0], (0, n_pad - n)).reshape(rows, 128)
    c2 = pl.pallas_call(
        _c_tc_body,
        out_shape=jax.ShapeDtypeStruct((rows, 128), jnp.float32),
    )(mu2, ls2, eps2)
    c = c2.reshape(-1)
    partials = _make_sc_loglik(n_edges, n_pad)(edges.reshape(-1), c)
    return jnp.sum(partials)


# speedup vs baseline: 654.6094x; 44.6655x over previous
import functools

import jax
import jax.numpy as jnp
from jax import lax
from jax.experimental import pallas as pl
from jax.experimental.pallas import tpu as pltpu
from jax.experimental.pallas import tpu_sc as plsc

NC = 2
NS = 16
NW = NC * NS
L = 16
BLK = 128

LN2 = 0.6931471805599453
SQRT2 = 1.4142135381698608


def _c_tc_body(mu_ref, ls_ref, eps_ref, c_ref):
    c_ref[...] = mu_ref[...] + jnp.exp(ls_ref[...]) * eps_ref[...]


def _log_f32(y):
    yi = plsc.bitcast(y, jnp.int32)
    e = lax.shift_right_logical(yi, 23) - 127
    m = plsc.bitcast((yi & 0x007FFFFF) | 0x3F800000, jnp.float32)
    big = m >= SQRT2
    m = jnp.where(big, m * 0.5, m)
    ef = (e + jnp.where(big, 1, 0)).astype(jnp.float32)
    s = (m - 1.0) / (m + 1.0)
    z = s * s
    p = s * (2.0 + z * (0.66666667 + z * (0.4 + z * 0.28571429)))
    return ef * LN2 + p


def _edge_loglik_group(ebuf, off_i, c_v, acc):
    ii = ebuf[pl.ds(off_i, L)]
    jj = ebuf[pl.ds(off_i + BLK, L)]
    ci = plsc.load_gather(c_v, [ii])
    cj = plsc.load_gather(c_v, [jj])
    x = ci * cj
    t = jnp.exp(-x)
    y = 1.0 / (1.0 + t) + 1e-8
    return acc + _log_f32(y)


def _make_sc_loglik(n_edges, n_pad):
    assert n_edges % BLK == 0
    n_blocks = n_edges // BLK
    bpt = n_blocks // NW
    n_extra = n_blocks - bpt * NW
    cblk = 1
    for cand in range(24, 1, -1):
        if bpt % cand == 0:
            cblk = cand
            break
    n_chunks = bpt // cblk
    cwords = cblk * 2 * BLK

    mesh = plsc.VectorSubcoreMesh(
        core_axis_name="c", subcore_axis_name="s",
        num_cores=NC, num_subcores=NS)

    @functools.partial(
        pl.kernel,
        out_type=jax.ShapeDtypeStruct((NW, L), jnp.float32),
        mesh=mesh,
        compiler_params=pltpu.CompilerParams(needs_layout_passes=False),
        scratch_types=[
            pltpu.VMEM((n_pad,), jnp.float32),
            pltpu.VMEM((cwords,), jnp.int32),
            pltpu.VMEM((L,), jnp.float32),
        ],
    )
    def sc_loglik(edges_hbm, c_hbm, out_hbm, c_v, ebuf, acc_v):
        cid = lax.axis_index("c")
        sid = lax.axis_index("s")
        wid = sid * NC + cid
        pltpu.sync_copy(c_hbm, c_v)
        base_word = wid * (bpt * 2 * BLK)

        def grp_body(t, acc):
            off = (t >> 3) * (2 * BLK) + (t & 7) * L
            return _edge_loglik_group(ebuf, off, c_v, acc)

        def chunk_body(k, acc):
            w0 = pl.multiple_of(base_word + k * cwords, 8)
            pltpu.sync_copy(edges_hbm.at[pl.ds(w0, cwords)], ebuf)
            return lax.fori_loop(0, cblk * (BLK // L), grp_body, acc)

        acc = lax.fori_loop(0, n_chunks, chunk_body,
                            jnp.zeros((L,), jnp.float32))

        if n_extra:
            has_extra = wid < n_extra
            bex = NW * bpt + jnp.where(has_extra, wid, 0) - \
                jnp.where(has_extra, 0, n_extra)
            wex = pl.multiple_of(bex * (2 * BLK), 8)
            pltpu.sync_copy(edges_hbm.at[pl.ds(wex, 2 * BLK)],
                            ebuf.at[pl.ds(0, 2 * BLK)])
            eacc = lax.fori_loop(
                0, BLK // L,
                lambda g, a: _edge_loglik_group(ebuf, g * L, c_v, a),
                jnp.zeros((L,), jnp.float32))
            acc = acc + jnp.where(has_extra, 1.0, 0.0) * eacc

        acc_v[...] = acc
        pltpu.sync_copy(acc_v, out_hbm.at[wid])

    return sc_loglik


def kernel(edges, n_samples, eps, mu, log_sigma):
    n = mu.shape[0]
    n_edges = edges.shape[0]
    n_pad = ((n + 1023) // 1024) * 1024
    rows = n_pad // 128
    mu2 = jnp.pad(mu, (0, n_pad - n)).reshape(rows, 128)
    ls2 = jnp.pad(log_sigma, (0, n_pad - n)).reshape(rows, 128)
    eps2 = jnp.pad(eps[0], (0, n_pad - n)).reshape(rows, 128)
    c2 = pl.pallas_call(
        _c_tc_body,
        out_shape=jax.ShapeDtypeStruct((rows, 128), jnp.float32),
    )(mu2, ls2, eps2)
    c = c2.reshape(-1)
    ef = edges.reshape(n_edges // BLK, BLK, 2).transpose(0, 2, 1).reshape(-1)
    partials = _make_sc_loglik(n_edges, n_pad)(ef, c)
    return jnp.sum(partials)


# speedup vs baseline: 942.8058x; 1.4403x over previous
import functools

import jax
import jax.numpy as jnp
from jax import lax
from jax.experimental import pallas as pl
from jax.experimental.pallas import tpu as pltpu
from jax.experimental.pallas import tpu_sc as plsc

NC = 2
NS = 16
NW = NC * NS
L = 16
BLK = 128

LN2 = 0.6931471805599453


def _c_tc_body(mu_ref, ls_ref, eps_ref, c_ref):
    c_ref[...] = mu_ref[...] + jnp.exp(ls_ref[...]) * eps_ref[...]


def _edge_group(ebuf, off, c_v, acc):
    ii = ebuf[pl.ds(off, L)]
    jj = ebuf[pl.ds(off + BLK, L)]
    ci = plsc.load_gather(c_v, [ii])
    cj = plsc.load_gather(c_v, [jj])
    x = ci * cj
    t = jnp.exp(jnp.minimum(-x, 87.0))
    d = 1.0 + t
    num = 1.00000001 + 1e-8 * t
    di = plsc.bitcast(d, jnp.int32)
    ni = plsc.bitcast(num, jnp.int32)
    de = lax.shift_right_logical(ni, 23) - lax.shift_right_logical(di, 23)
    md = plsc.bitcast((di & 0x007FFFFF) | 0x3F800000, jnp.float32)
    mn = plsc.bitcast((ni & 0x007FFFFF) | 0x3F800000, jnp.float32)
    u = (mn - md) / (mn + md)
    z = u * u
    poly = u * (2.0 + z * (0.66666667 + z * (0.4 + z * 0.28571429)))
    return acc + (de.astype(jnp.float32) * LN2 + poly)


def _make_sc_loglik(n_edges, n_pad):
    assert n_edges % BLK == 0
    n_blocks = n_edges // BLK
    bpt = n_blocks // NW
    n_extra = n_blocks - bpt * NW
    cblk = 1
    for cand in range(16, 1, -1):
        if bpt % cand == 0 and (bpt // cand) % 2 == 0:
            cblk = cand
            break
    n_chunks = bpt // cblk
    assert n_chunks % 2 == 0
    cwords = cblk * 2 * BLK
    gpairs = cblk * (BLK // L) // 2

    mesh = plsc.VectorSubcoreMesh(
        core_axis_name="c", subcore_axis_name="s",
        num_cores=NC, num_subcores=NS)

    @functools.partial(
        pl.kernel,
        out_type=jax.ShapeDtypeStruct((NW, L), jnp.float32),
        mesh=mesh,
        compiler_params=pltpu.CompilerParams(needs_layout_passes=False),
        scratch_types=[
            pltpu.VMEM((n_pad,), jnp.float32),
            pltpu.VMEM((cwords,), jnp.int32),
            pltpu.VMEM((cwords,), jnp.int32),
            pltpu.VMEM((L,), jnp.float32),
            pltpu.SemaphoreType.DMA,
            pltpu.SemaphoreType.DMA,
        ],
    )
    def sc_loglik(edges_hbm, c_hbm, out_hbm, c_v, ebuf0, ebuf1, acc_v,
                  sem0, sem1):
        cid = lax.axis_index("c")
        sid = lax.axis_index("s")
        wid = sid * NC + cid
        base_word = wid * (bpt * 2 * BLK)

        def chunk_src(k):
            w0 = pl.multiple_of(base_word + k * cwords, 8)
            return edges_hbm.at[pl.ds(w0, cwords)]

        pltpu.make_async_copy(chunk_src(0), ebuf0, sem0).start()
        pltpu.make_async_copy(chunk_src(1), ebuf1, sem1).start()
        pltpu.sync_copy(c_hbm, c_v)

        def process_chunk(ebuf, accs):
            def gp_body(g, a):
                a0, a1 = a
                g0 = 2 * g
                off0 = (g0 >> 3) * (2 * BLK) + (g0 & 7) * L
                g1 = g0 + 1
                off1 = (g1 >> 3) * (2 * BLK) + (g1 & 7) * L
                return (_edge_group(ebuf, off0, c_v, a0),
                        _edge_group(ebuf, off1, c_v, a1))
            return lax.fori_loop(0, gpairs, gp_body, accs)

        def pair_body(k, accs):
            pltpu.make_async_copy(chunk_src(2 * k), ebuf0, sem0).wait()
            accs = process_chunk(ebuf0, accs)

            @pl.when(k < n_chunks // 2 - 1)
            def _():
                pltpu.make_async_copy(chunk_src(2 * k + 2), ebuf0,
                                      sem0).start()

            pltpu.make_async_copy(chunk_src(2 * k + 1), ebuf1, sem1).wait()
            accs = process_chunk(ebuf1, accs)

            @pl.when(k < n_chunks // 2 - 1)
            def _():
                pltpu.make_async_copy(chunk_src(2 * k + 3), ebuf1,
                                      sem1).start()

            return accs

        zero = jnp.zeros((L,), jnp.float32)
        a0, a1 = lax.fori_loop(0, n_chunks // 2, pair_body, (zero, zero))

        if n_extra:
            has_extra = wid < n_extra
            bex = NW * bpt + jnp.where(has_extra, wid, 0) - \
                jnp.where(has_extra, 0, n_extra)
            wex = pl.multiple_of(bex * (2 * BLK), 8)
            pltpu.sync_copy(edges_hbm.at[pl.ds(wex, 2 * BLK)],
                            ebuf0.at[pl.ds(0, 2 * BLK)])
            e0, e1 = lax.fori_loop(
                0, BLK // L // 2,
                lambda g, a: (_edge_group(ebuf0, (2 * g) * L, c_v, a[0]),
                              _edge_group(ebuf0, (2 * g + 1) * L, c_v, a[1])),
                (zero, zero))
            scale = jnp.where(has_extra, 1.0, 0.0)
            a0 = a0 + scale * e0
            a1 = a1 + scale * e1

        acc_v[...] = a0 + a1
        pltpu.sync_copy(acc_v, out_hbm.at[wid])

    return sc_loglik


def kernel(edges, n_samples, eps, mu, log_sigma):
    n = mu.shape[0]
    n_edges = edges.shape[0]
    n_pad = ((n + 1023) // 1024) * 1024
    rows = n_pad // 128
    mu2 = jnp.pad(mu, (0, n_pad - n)).reshape(rows, 128)
    ls2 = jnp.pad(log_sigma, (0, n_pad - n)).reshape(rows, 128)
    eps2 = jnp.pad(eps[0], (0, n_pad - n)).reshape(rows, 128)
    c2 = pl.pallas_call(
        _c_tc_body,
        out_shape=jax.ShapeDtypeStruct((rows, 128), jnp.float32),
    )(mu2, ls2, eps2)
    c = c2.reshape(-1)
    ef = edges.reshape(n_edges // BLK, BLK, 2).transpose(0, 2, 1).reshape(-1)
    partials = _make_sc_loglik(n_edges, n_pad)(ef, c)
    return jnp.sum(partials)


# speedup vs baseline: 980.7886x; 1.0403x over previous
import functools

import jax
import jax.numpy as jnp
from jax import lax
from jax.experimental import pallas as pl
from jax.experimental.pallas import tpu as pltpu
from jax.experimental.pallas import tpu_sc as plsc

NC = 2
NS = 16
NW = NC * NS
L = 16
BLK = 128

LN2 = 0.6931471805599453


def _c_tc_body(mu_ref, ls_ref, eps_ref, c_ref):
    c_ref[...] = mu_ref[...] + jnp.exp(ls_ref[...]) * eps_ref[...]


def _edge_group(ebuf, off, c_v, acc):
    accf, acci = acc
    ii = ebuf[pl.ds(off, L)]
    jj = ebuf[pl.ds(off + BLK, L)]
    ci = plsc.load_gather(c_v, [ii])
    cj = plsc.load_gather(c_v, [jj])
    x = ci * cj
    t = jnp.exp(jnp.minimum(-x, 87.0))
    d = 1.0 + t
    num = 1.00000001 + 1e-8 * t
    di = plsc.bitcast(d, jnp.int32)
    ni = plsc.bitcast(num, jnp.int32)
    de = lax.shift_right_logical(ni, 23) - lax.shift_right_logical(di, 23)
    md = plsc.bitcast((di & 0x007FFFFF) | 0x3F800000, jnp.float32)
    mn = plsc.bitcast((ni & 0x007FFFFF) | 0x3F800000, jnp.float32)
    u = (mn - md) / (mn + md)
    z = u * u
    poly = u * (2.0 + z * (0.66666667 + z * 0.4))
    return accf + poly, acci + de


def _make_sc_loglik(n_edges, n_pad):
    assert n_edges % BLK == 0
    n_blocks = n_edges // BLK
    bpt = n_blocks // NW
    n_extra = n_blocks - bpt * NW
    cblk = 1
    for cand in range(16, 1, -1):
        if bpt % cand == 0 and (bpt // cand) % 2 == 0:
            cblk = cand
            break
    n_chunks = bpt // cblk
    assert n_chunks % 2 == 0
    cwords = cblk * 2 * BLK
    gquads = cblk * (BLK // L) // 4

    mesh = plsc.VectorSubcoreMesh(
        core_axis_name="c", subcore_axis_name="s",
        num_cores=NC, num_subcores=NS)

    @functools.partial(
        pl.kernel,
        out_type=jax.ShapeDtypeStruct((NW, L), jnp.float32),
        mesh=mesh,
        compiler_params=pltpu.CompilerParams(needs_layout_passes=False),
        scratch_types=[
            pltpu.VMEM((n_pad,), jnp.float32),
            pltpu.VMEM((cwords,), jnp.int32),
            pltpu.VMEM((cwords,), jnp.int32),
            pltpu.VMEM((L,), jnp.float32),
            pltpu.SemaphoreType.DMA,
            pltpu.SemaphoreType.DMA,
        ],
    )
    def sc_loglik(edges_hbm, c_hbm, out_hbm, c_v, ebuf0, ebuf1, acc_v,
                  sem0, sem1):
        cid = lax.axis_index("c")
        sid = lax.axis_index("s")
        wid = sid * NC + cid
        base_word = wid * (bpt * 2 * BLK)

        def chunk_src(k):
            w0 = pl.multiple_of(base_word + k * cwords, 8)
            return edges_hbm.at[pl.ds(w0, cwords)]

        pltpu.make_async_copy(chunk_src(0), ebuf0, sem0).start()
        pltpu.make_async_copy(chunk_src(1), ebuf1, sem1).start()
        pltpu.sync_copy(c_hbm, c_v)

        def process_chunk(ebuf, accs):
            def gp_body(q, a):
                a0, a1, a2, a3 = a
                ob = (q >> 1) * (2 * BLK) + (q & 1) * (4 * L)
                return (_edge_group(ebuf, ob, c_v, a0),
                        _edge_group(ebuf, ob + L, c_v, a1),
                        _edge_group(ebuf, ob + 2 * L, c_v, a2),
                        _edge_group(ebuf, ob + 3 * L, c_v, a3))
            return lax.fori_loop(0, gquads, gp_body, accs)

        def pair_body(k, accs):
            pltpu.make_async_copy(chunk_src(2 * k), ebuf0, sem0).wait()
            accs = process_chunk(ebuf0, accs)

            @pl.when(k < n_chunks // 2 - 1)
            def _():
                pltpu.make_async_copy(chunk_src(2 * k + 2), ebuf0,
                                      sem0).start()

            pltpu.make_async_copy(chunk_src(2 * k + 1), ebuf1, sem1).wait()
            accs = process_chunk(ebuf1, accs)

            @pl.when(k < n_chunks // 2 - 1)
            def _():
                pltpu.make_async_copy(chunk_src(2 * k + 3), ebuf1,
                                      sem1).start()

            return accs

        zf = jnp.zeros((L,), jnp.float32)
        zi = jnp.zeros((L,), jnp.int32)
        zero4 = ((zf, zi),) * 4
        a0, a1, a2, a3 = lax.fori_loop(0, n_chunks // 2, pair_body, zero4)

        if n_extra:
            has_extra = wid < n_extra
            bex = NW * bpt + jnp.where(has_extra, wid, 0) - \
                jnp.where(has_extra, 0, n_extra)
            wex = pl.multiple_of(bex * (2 * BLK), 8)
            pltpu.sync_copy(edges_hbm.at[pl.ds(wex, 2 * BLK)],
                            ebuf0.at[pl.ds(0, 2 * BLK)])
            e0, e1, e2, e3 = lax.fori_loop(
                0, BLK // L // 4,
                lambda q, a: (_edge_group(ebuf0, (4 * q) * L, c_v, a[0]),
                              _edge_group(ebuf0, (4 * q + 1) * L, c_v, a[1]),
                              _edge_group(ebuf0, (4 * q + 2) * L, c_v, a[2]),
                              _edge_group(ebuf0, (4 * q + 3) * L, c_v, a[3])),
                zero4)
            ef32 = (e0[0] + e1[0]) + (e2[0] + e3[0])
            ei32 = (e0[1] + e1[1]) + (e2[1] + e3[1])
            scale = jnp.where(has_extra, 1.0, 0.0)
            a0 = (a0[0] + scale * ef32,
                  a0[1] + jnp.where(has_extra, ei32, 0))

        accf = (a0[0] + a1[0]) + (a2[0] + a3[0])
        acci = (a0[1] + a1[1]) + (a2[1] + a3[1])
        acc_v[...] = accf + acci.astype(jnp.float32) * LN2
        pltpu.sync_copy(acc_v, out_hbm.at[wid])

    return sc_loglik


def kernel(edges, n_samples, eps, mu, log_sigma):
    n = mu.shape[0]
    n_edges = edges.shape[0]
    n_pad = ((n + 1023) // 1024) * 1024
    rows = n_pad // 128
    mu2 = jnp.pad(mu, (0, n_pad - n)).reshape(rows, 128)
    ls2 = jnp.pad(log_sigma, (0, n_pad - n)).reshape(rows, 128)
    eps2 = jnp.pad(eps[0], (0, n_pad - n)).reshape(rows, 128)
    c2 = pl.pallas_call(
        _c_tc_body,
        out_shape=jax.ShapeDtypeStruct((rows, 128), jnp.float32),
    )(mu2, ls2, eps2)
    c = c2.reshape(-1)
    ef = edges.reshape(n_edges // BLK, BLK, 2).transpose(0, 2, 1).reshape(-1)
    partials = _make_sc_loglik(n_edges, n_pad)(ef, c)
    return jnp.sum(partials)


# speedup vs baseline: 1037.7888x; 1.0581x over previous
import functools

import jax
import jax.numpy as jnp
from jax import lax
from jax.experimental import pallas as pl
from jax.experimental.pallas import tpu as pltpu
from jax.experimental.pallas import tpu_sc as plsc

NC = 2
NS = 16
NW = NC * NS
L = 16
BLK = 128

TAB_SCALE = 32.0
TAB_BIAS = 1088.0
TAB_N = 2177
TAB_PAD = 3072


def _c_tc_body(mu_ref, ls_ref, eps_ref, c_ref, tab_ref):
    c_ref[...] = mu_ref[...] + jnp.exp(ls_ref[...]) * eps_ref[...]
    rows, cols = tab_ref.shape
    r = lax.broadcasted_iota(jnp.int32, (rows, cols), 0)
    cc = lax.broadcasted_iota(jnp.int32, (rows, cols), 1)
    flat = jnp.minimum(r * cols + cc, TAB_N - 1).astype(jnp.float32)
    xs = (flat - TAB_BIAS) * (1.0 / TAB_SCALE)
    tab_ref[...] = jnp.log(jax.nn.sigmoid(xs) + 1e-8)


def _edge_group(ebuf, off, c_v, tab_v, acc):
    ii = ebuf[pl.ds(off, L)]
    jj = ebuf[pl.ds(off + BLK, L)]
    ci = plsc.load_gather(c_v, [ii])
    cj = plsc.load_gather(c_v, [jj])
    x = ci * cj
    v = x * TAB_SCALE + TAB_BIAS
    v = jnp.minimum(jnp.maximum(v, 0.0), TAB_N - 1.001)
    k = v.astype(jnp.int32)
    frac = v - k.astype(jnp.float32)
    t0 = plsc.load_gather(tab_v, [k])
    t1 = plsc.load_gather(tab_v, [k + 1])
    return acc + (t0 + frac * (t1 - t0))


def _make_sc_loglik(n_edges, n_pad):
    assert n_edges % BLK == 0
    n_blocks = n_edges // BLK
    bpt = n_blocks // NW
    n_extra = n_blocks - bpt * NW
    cblk = 1
    for cand in range(16, 1, -1):
        if bpt % cand == 0 and (bpt // cand) % 2 == 0:
            cblk = cand
            break
    n_chunks = bpt // cblk
    assert n_chunks % 2 == 0
    cwords = cblk * 2 * BLK
    gquads = cblk * (BLK // L) // 4

    mesh = plsc.VectorSubcoreMesh(
        core_axis_name="c", subcore_axis_name="s",
        num_cores=NC, num_subcores=NS)

    @functools.partial(
        pl.kernel,
        out_type=jax.ShapeDtypeStruct((NW, L), jnp.float32),
        mesh=mesh,
        compiler_params=pltpu.CompilerParams(needs_layout_passes=False),
        scratch_types=[
            pltpu.VMEM((n_pad,), jnp.float32),
            pltpu.VMEM((TAB_PAD,), jnp.float32),
            pltpu.VMEM((cwords,), jnp.int32),
            pltpu.VMEM((cwords,), jnp.int32),
            pltpu.VMEM((L,), jnp.float32),
            pltpu.SemaphoreType.DMA,
            pltpu.SemaphoreType.DMA,
        ],
    )
    def sc_loglik(edges_hbm, c_hbm, tab_hbm, out_hbm, c_v, tab_v,
                  ebuf0, ebuf1, acc_v, sem0, sem1):
        cid = lax.axis_index("c")
        sid = lax.axis_index("s")
        wid = sid * NC + cid
        base_word = wid * (bpt * 2 * BLK)

        def chunk_src(k):
            w0 = pl.multiple_of(base_word + k * cwords, 8)
            return edges_hbm.at[pl.ds(w0, cwords)]

        pltpu.make_async_copy(chunk_src(0), ebuf0, sem0).start()
        pltpu.make_async_copy(chunk_src(1), ebuf1, sem1).start()
        pltpu.sync_copy(c_hbm, c_v)
        pltpu.sync_copy(tab_hbm, tab_v)

        def process_chunk(ebuf, accs):
            def gp_body(q, a):
                a0, a1, a2, a3 = a
                ob = (q >> 1) * (2 * BLK) + (q & 1) * (4 * L)
                return (_edge_group(ebuf, ob, c_v, tab_v, a0),
                        _edge_group(ebuf, ob + L, c_v, tab_v, a1),
                        _edge_group(ebuf, ob + 2 * L, c_v, tab_v, a2),
                        _edge_group(ebuf, ob + 3 * L, c_v, tab_v, a3))
            return lax.fori_loop(0, gquads, gp_body, accs)

        def pair_body(k, accs):
            pltpu.make_async_copy(chunk_src(2 * k), ebuf0, sem0).wait()
            accs = process_chunk(ebuf0, accs)

            @pl.when(k < n_chunks // 2 - 1)
            def _():
                pltpu.make_async_copy(chunk_src(2 * k + 2), ebuf0,
                                      sem0).start()

            pltpu.make_async_copy(chunk_src(2 * k + 1), ebuf1, sem1).wait()
            accs = process_chunk(ebuf1, accs)

            @pl.when(k < n_chunks // 2 - 1)
            def _():
                pltpu.make_async_copy(chunk_src(2 * k + 3), ebuf1,
                                      sem1).start()

            return accs

        zf = jnp.zeros((L,), jnp.float32)
        zero4 = (zf,) * 4
        a0, a1, a2, a3 = lax.fori_loop(0, n_chunks // 2, pair_body, zero4)

        if n_extra:
            has_extra = wid < n_extra
            bex = NW * bpt + jnp.where(has_extra, wid, 0) - \
                jnp.where(has_extra, 0, n_extra)
            wex = pl.multiple_of(bex * (2 * BLK), 8)
            pltpu.sync_copy(edges_hbm.at[pl.ds(wex, 2 * BLK)],
                            ebuf0.at[pl.ds(0, 2 * BLK)])
            e0, e1, e2, e3 = lax.fori_loop(
                0, BLK // L // 4,
                lambda q, a: tuple(
                    _edge_group(ebuf0, (4 * q + i) * L, c_v, tab_v, a[i])
                    for i in range(4)),
                zero4)
            scale = jnp.where(has_extra, 1.0, 0.0)
            a0 = a0 + scale * ((e0 + e1) + (e2 + e3))

        acc_v[...] = (a0 + a1) + (a2 + a3)
        pltpu.sync_copy(acc_v, out_hbm.at[wid])

    return sc_loglik


def kernel(edges, n_samples, eps, mu, log_sigma):
    n = mu.shape[0]
    n_edges = edges.shape[0]
    n_pad = ((n + 1023) // 1024) * 1024
    rows = n_pad // 128
    mu2 = jnp.pad(mu, (0, n_pad - n)).reshape(rows, 128)
    ls2 = jnp.pad(log_sigma, (0, n_pad - n)).reshape(rows, 128)
    eps2 = jnp.pad(eps[0], (0, n_pad - n)).reshape(rows, 128)
    c2, tab2 = pl.pallas_call(
        _c_tc_body,
        out_shape=(jax.ShapeDtypeStruct((rows, 128), jnp.float32),
                   jax.ShapeDtypeStruct((TAB_PAD // 128, 128), jnp.float32)),
    )(mu2, ls2, eps2)
    c = c2.reshape(-1)
    tab = tab2.reshape(-1)
    ef = edges.reshape(n_edges // BLK, BLK, 2).transpose(0, 2, 1).reshape(-1)
    partials = _make_sc_loglik(n_edges, n_pad)(ef, c, tab)
    return jnp.sum(partials)


# speedup vs baseline: 1041.4931x; 1.0036x over previous
import functools

import jax
import jax.numpy as jnp
from jax import lax
from jax.experimental import pallas as pl
from jax.experimental.pallas import tpu as pltpu
from jax.experimental.pallas import tpu_sc as plsc

NC = 2
NS = 16
NW = NC * NS
L = 16
BLK = 128

TAB_SCALE = 32.0
TAB_BIAS = 1088.0
TAB_N = 2177
TAB_PAD = 3072


def _c_tc_body(mu_ref, ls_ref, eps_ref, c_ref, tab_ref):
    c_ref[...] = mu_ref[...] + jnp.exp(ls_ref[...]) * eps_ref[...]
    rows, cols = tab_ref.shape
    r = lax.broadcasted_iota(jnp.int32, (rows, cols), 0)
    cc = lax.broadcasted_iota(jnp.int32, (rows, cols), 1)
    flat = jnp.minimum(r * cols + cc, TAB_N - 1).astype(jnp.float32)
    xs = (flat - TAB_BIAS) * (1.0 / TAB_SCALE)
    tab_ref[...] = jnp.log(jax.nn.sigmoid(xs) + 1e-8)


def _edge_group(ebuf, off, c_v, tab_v, acc):
    ii = ebuf[pl.ds(off, L)]
    jj = ebuf[pl.ds(off + BLK, L)]
    ci = plsc.load_gather(c_v, [ii])
    cj = plsc.load_gather(c_v, [jj])
    x = ci * cj
    v = x * TAB_SCALE + TAB_BIAS
    v = jnp.minimum(jnp.maximum(v, 0.0), TAB_N - 1.001)
    k = v.astype(jnp.int32)
    frac = v - k.astype(jnp.float32)
    t0 = plsc.load_gather(tab_v, [k])
    t1 = plsc.load_gather(tab_v, [k + 1])
    return acc + (t0 + frac * (t1 - t0))


def _make_sc_loglik(n_edges, n_pad):
    assert n_edges % BLK == 0
    n_blocks = n_edges // BLK
    bpt = n_blocks // NW
    n_extra = n_blocks - bpt * NW
    cblk = 1
    for cand in range(16, 1, -1):
        if bpt % cand == 0 and (bpt // cand) % 2 == 0:
            cblk = cand
            break
    n_chunks = bpt // cblk
    assert n_chunks % 2 == 0
    cwords = cblk * 2 * BLK
    gquads = cblk * (BLK // L) // 4

    mesh = plsc.VectorSubcoreMesh(
        core_axis_name="c", subcore_axis_name="s",
        num_cores=NC, num_subcores=NS)

    @functools.partial(
        pl.kernel,
        out_type=jax.ShapeDtypeStruct((NW, L), jnp.float32),
        mesh=mesh,
        compiler_params=pltpu.CompilerParams(needs_layout_passes=False),
        scratch_types=[
            pltpu.VMEM((n_pad,), jnp.float32),
            pltpu.VMEM((TAB_PAD,), jnp.float32),
            pltpu.VMEM((cwords,), jnp.int32),
            pltpu.VMEM((cwords,), jnp.int32),
            pltpu.VMEM((L,), jnp.float32),
            pltpu.SemaphoreType.DMA,
            pltpu.SemaphoreType.DMA,
        ],
    )
    def sc_loglik(edges_hbm, c_hbm, tab_hbm, out_hbm, c_v, tab_v,
                  ebuf0, ebuf1, acc_v, sem0, sem1):
        cid = lax.axis_index("c")
        sid = lax.axis_index("s")
        wid = sid * NC + cid
        base_word = wid * (bpt * 2 * BLK)

        def chunk_src(k):
            w0 = pl.multiple_of(base_word + k * cwords, 8)
            return edges_hbm.at[pl.ds(w0, cwords)]

        pltpu.make_async_copy(chunk_src(0), ebuf0, sem0).start()
        pltpu.make_async_copy(chunk_src(1), ebuf1, sem1).start()
        pltpu.sync_copy(c_hbm, c_v)
        pltpu.sync_copy(tab_hbm, tab_v)

        def process_chunk(ebuf, accs):
            def gp_body(q, a):
                a0, a1, a2, a3 = a
                ob = (q >> 1) * (2 * BLK) + (q & 1) * (4 * L)
                return (_edge_group(ebuf, ob, c_v, tab_v, a0),
                        _edge_group(ebuf, ob + L, c_v, tab_v, a1),
                        _edge_group(ebuf, ob + 2 * L, c_v, tab_v, a2),
                        _edge_group(ebuf, ob + 3 * L, c_v, tab_v, a3))
            return plsc.parallel_loop(0, gquads, unroll=2,
                                      carry=accs)(gp_body)

        def pair_body(k, accs):
            pltpu.make_async_copy(chunk_src(2 * k), ebuf0, sem0).wait()
            accs = process_chunk(ebuf0, accs)

            @pl.when(k < n_chunks // 2 - 1)
            def _():
                pltpu.make_async_copy(chunk_src(2 * k + 2), ebuf0,
                                      sem0).start()

            pltpu.make_async_copy(chunk_src(2 * k + 1), ebuf1, sem1).wait()
            accs = process_chunk(ebuf1, accs)

            @pl.when(k < n_chunks // 2 - 1)
            def _():
                pltpu.make_async_copy(chunk_src(2 * k + 3), ebuf1,
                                      sem1).start()

            return accs

        zf = jnp.zeros((L,), jnp.float32)
        zero4 = (zf,) * 4
        a0, a1, a2, a3 = lax.fori_loop(0, n_chunks // 2, pair_body, zero4)

        if n_extra:
            has_extra = wid < n_extra
            bex = NW * bpt + jnp.where(has_extra, wid, 0) - \
                jnp.where(has_extra, 0, n_extra)
            wex = pl.multiple_of(bex * (2 * BLK), 8)
            pltpu.sync_copy(edges_hbm.at[pl.ds(wex, 2 * BLK)],
                            ebuf0.at[pl.ds(0, 2 * BLK)])
            e0, e1, e2, e3 = lax.fori_loop(
                0, BLK // L // 4,
                lambda q, a: tuple(
                    _edge_group(ebuf0, (4 * q + i) * L, c_v, tab_v, a[i])
                    for i in range(4)),
                zero4)
            scale = jnp.where(has_extra, 1.0, 0.0)
            a0 = a0 + scale * ((e0 + e1) + (e2 + e3))

        acc_v[...] = (a0 + a1) + (a2 + a3)
        pltpu.sync_copy(acc_v, out_hbm.at[wid])

    return sc_loglik


def kernel(edges, n_samples, eps, mu, log_sigma):
    n = mu.shape[0]
    n_edges = edges.shape[0]
    n_pad = ((n + 1023) // 1024) * 1024
    rows = n_pad // 128
    mu2 = jnp.pad(mu, (0, n_pad - n)).reshape(rows, 128)
    ls2 = jnp.pad(log_sigma, (0, n_pad - n)).reshape(rows, 128)
    eps2 = jnp.pad(eps[0], (0, n_pad - n)).reshape(rows, 128)
    c2, tab2 = pl.pallas_call(
        _c_tc_body,
        out_shape=(jax.ShapeDtypeStruct((rows, 128), jnp.float32),
                   jax.ShapeDtypeStruct((TAB_PAD // 128, 128), jnp.float32)),
    )(mu2, ls2, eps2)
    c = c2.reshape(-1)
    tab = tab2.reshape(-1)
    ef = edges.reshape(n_edges // BLK, BLK, 2).transpose(0, 2, 1).reshape(-1)
    partials = _make_sc_loglik(n_edges, n_pad)(ef, c, tab)
    return jnp.sum(partials)


# speedup vs baseline: 1067.0392x; 1.0245x over previous
import functools

import jax
import jax.numpy as jnp
from jax import lax
from jax.experimental import pallas as pl
from jax.experimental.pallas import tpu as pltpu
from jax.experimental.pallas import tpu_sc as plsc

NC = 2
NS = 16
NW = NC * NS
L = 16
BLK = 128

TAB_SCALE = 128.0
TAB_BIAS = 4352.0
TAB_N = 8705
TAB_PAD = 9216


def _c_tc_body(mu_ref, ls_ref, eps_ref, c_ref, tab_ref):
    c_ref[...] = mu_ref[...] + jnp.exp(ls_ref[...]) * eps_ref[...]
    rows, cols = tab_ref.shape
    r = lax.broadcasted_iota(jnp.int32, (rows, cols), 0)
    cc = lax.broadcasted_iota(jnp.int32, (rows, cols), 1)
    flat = jnp.minimum(r * cols + cc, TAB_N - 1).astype(jnp.float32)
    xs = (flat - TAB_BIAS) * (1.0 / TAB_SCALE)
    tab_ref[...] = jnp.log(jax.nn.sigmoid(xs) + 1e-8)


def _edge_group(ebuf, off, c_v, tab_v, acc):
    ii = ebuf[pl.ds(off, L)]
    jj = ebuf[pl.ds(off + BLK, L)]
    ci = plsc.load_gather(c_v, [ii])
    cj = plsc.load_gather(c_v, [jj])
    x = ci * cj
    v = x * TAB_SCALE + (TAB_BIAS + 0.5)
    v = jnp.minimum(jnp.maximum(v, 0.0), TAB_N - 1.0)
    k = v.astype(jnp.int32)
    return acc + plsc.load_gather(tab_v, [k])


def _make_sc_loglik(n_edges, n_pad):
    assert n_edges % BLK == 0
    n_blocks = n_edges // BLK
    bpt = n_blocks // NW
    n_extra = n_blocks - bpt * NW
    cblk = 1
    for cand in range(16, 1, -1):
        if bpt % cand == 0 and (bpt // cand) % 2 == 0:
            cblk = cand
            break
    n_chunks = bpt // cblk
    assert n_chunks % 2 == 0
    cwords = cblk * 2 * BLK
    gquads = cblk * (BLK // L) // 4

    mesh = plsc.VectorSubcoreMesh(
        core_axis_name="c", subcore_axis_name="s",
        num_cores=NC, num_subcores=NS)

    @functools.partial(
        pl.kernel,
        out_type=jax.ShapeDtypeStruct((NW, L), jnp.float32),
        mesh=mesh,
        compiler_params=pltpu.CompilerParams(needs_layout_passes=False),
        scratch_types=[
            pltpu.VMEM((n_pad,), jnp.float32),
            pltpu.VMEM((TAB_PAD,), jnp.float32),
            pltpu.VMEM((cwords,), jnp.int32),
            pltpu.VMEM((cwords,), jnp.int32),
            pltpu.VMEM((L,), jnp.float32),
            pltpu.SemaphoreType.DMA,
            pltpu.SemaphoreType.DMA,
        ],
    )
    def sc_loglik(edges_hbm, c_hbm, tab_hbm, out_hbm, c_v, tab_v,
                  ebuf0, ebuf1, acc_v, sem0, sem1):
        cid = lax.axis_index("c")
        sid = lax.axis_index("s")
        wid = sid * NC + cid
        base_word = wid * (bpt * 2 * BLK)

        def chunk_src(k):
            w0 = pl.multiple_of(base_word + k * cwords, 8)
            return edges_hbm.at[pl.ds(w0, cwords)]

        pltpu.make_async_copy(chunk_src(0), ebuf0, sem0).start()
        pltpu.make_async_copy(chunk_src(1), ebuf1, sem1).start()
        pltpu.sync_copy(c_hbm, c_v)
        pltpu.sync_copy(tab_hbm, tab_v)

        def process_chunk(ebuf, accs):
            def gp_body(q, a):
                a0, a1, a2, a3 = a
                ob = (q >> 1) * (2 * BLK) + (q & 1) * (4 * L)
                return (_edge_group(ebuf, ob, c_v, tab_v, a0),
                        _edge_group(ebuf, ob + L, c_v, tab_v, a1),
                        _edge_group(ebuf, ob + 2 * L, c_v, tab_v, a2),
                        _edge_group(ebuf, ob + 3 * L, c_v, tab_v, a3))
            return plsc.parallel_loop(0, gquads, unroll=2,
                                      carry=accs)(gp_body)

        def pair_body(k, accs):
            pltpu.make_async_copy(chunk_src(2 * k), ebuf0, sem0).wait()
            accs = process_chunk(ebuf0, accs)

            @pl.when(k < n_chunks // 2 - 1)
            def _():
                pltpu.make_async_copy(chunk_src(2 * k + 2), ebuf0,
                                      sem0).start()

            pltpu.make_async_copy(chunk_src(2 * k + 1), ebuf1, sem1).wait()
            accs = process_chunk(ebuf1, accs)

            @pl.when(k < n_chunks // 2 - 1)
            def _():
                pltpu.make_async_copy(chunk_src(2 * k + 3), ebuf1,
                                      sem1).start()

            return accs

        zf = jnp.zeros((L,), jnp.float32)
        zero4 = (zf,) * 4
        a0, a1, a2, a3 = lax.fori_loop(0, n_chunks // 2, pair_body, zero4)

        if n_extra:
            has_extra = wid < n_extra
            bex = NW * bpt + jnp.where(has_extra, wid, 0) - \
                jnp.where(has_extra, 0, n_extra)
            wex = pl.multiple_of(bex * (2 * BLK), 8)
            pltpu.sync_copy(edges_hbm.at[pl.ds(wex, 2 * BLK)],
                            ebuf0.at[pl.ds(0, 2 * BLK)])
            e0, e1, e2, e3 = lax.fori_loop(
                0, BLK // L // 4,
                lambda q, a: tuple(
                    _edge_group(ebuf0, (4 * q + i) * L, c_v, tab_v, a[i])
                    for i in range(4)),
                zero4)
            scale = jnp.where(has_extra, 1.0, 0.0)
            a0 = a0 + scale * ((e0 + e1) + (e2 + e3))

        acc_v[...] = (a0 + a1) + (a2 + a3)
        pltpu.sync_copy(acc_v, out_hbm.at[wid])

    return sc_loglik


def kernel(edges, n_samples, eps, mu, log_sigma):
    n = mu.shape[0]
    n_edges = edges.shape[0]
    n_pad = ((n + 1023) // 1024) * 1024
    rows = n_pad // 128
    mu2 = jnp.pad(mu, (0, n_pad - n)).reshape(rows, 128)
    ls2 = jnp.pad(log_sigma, (0, n_pad - n)).reshape(rows, 128)
    eps2 = jnp.pad(eps[0], (0, n_pad - n)).reshape(rows, 128)
    c2, tab2 = pl.pallas_call(
        _c_tc_body,
        out_shape=(jax.ShapeDtypeStruct((rows, 128), jnp.float32),
                   jax.ShapeDtypeStruct((TAB_PAD // 128, 128), jnp.float32)),
    )(mu2, ls2, eps2)
    c = c2.reshape(-1)
    tab = tab2.reshape(-1)
    ef = edges.reshape(n_edges // BLK, BLK, 2).transpose(0, 2, 1).reshape(-1)
    partials = _make_sc_loglik(n_edges, n_pad)(ef, c, tab)
    return jnp.sum(partials)
